# baseline (speedup 1.0000x reference)
import functools

import jax
import jax.numpy as jnp
from jax import lax
from jax.experimental import pallas as pl
from jax.experimental.pallas import tpu as pltpu
from jax.experimental.pallas import tpu_sc as plsc

N = 10000
E = 320000
H = 128
W = 3 * H

NC = 2
NS = 16
NW = NC * NS
K = 40
K2 = 40
G = 40
RPS = 624
NTAIL = N - NS * RPS

_mesh = plsc.VectorSubcoreMesh(core_axis_name="c", subcore_axis_name="s")


@functools.cache
def _make_edge_gather(ep):
    epw = ep // NW
    ch = epw // K

    @functools.partial(
        pl.kernel,
        mesh=_mesh,
        out_type=(
            jax.ShapeDtypeStruct((ep, 2 * H), jnp.float32),
            jax.ShapeDtypeStruct((ep, 16), jnp.float32),
        ),
        scratch_types=(
            pltpu.VMEM((epw,), jnp.int32),
            pltpu.VMEM((epw,), jnp.int32),
            pltpu.VMEM((2, K, W), jnp.float32),
            pltpu.VMEM((2, K, W), jnp.float32),
            pltpu.VMEM((2, K, 16), jnp.float32),
            pltpu.SemaphoreType.DMA((2,)),
            pltpu.SemaphoreType.DMA((2,)),
            pltpu.SemaphoreType.DMA((2,)),
        ),
    )
    def edge_gather(snd_hbm, rcv_hbm, send_hbm, rec_hbm, s_out, d_out,
                    sidx_all, ridx_all, buf_a, buf_b, dbuf,
                    sem_a, sem_b, sem_w):
        wid = lax.axis_index("s") * NC + lax.axis_index("c")
        e0 = wid * epw
        pltpu.sync_copy(send_hbm.at[pl.ds(e0, epw)], sidx_all)
        pltpu.sync_copy(rec_hbm.at[pl.ds(e0, epw)], ridx_all)

        def fire_gather(t, b):
            off = t * K
            pltpu.async_copy(snd_hbm.at[sidx_all.at[pl.ds(off, K)]],
                             buf_a.at[b], sem_a.at[b])
            pltpu.async_copy(rcv_hbm.at[ridx_all.at[pl.ds(off, K)]],
                             buf_b.at[b], sem_b.at[b])

        def wait_gather(t, b):
            off = t * K
            pltpu.make_async_copy(snd_hbm.at[sidx_all.at[pl.ds(off, K)]],
                                  buf_a.at[b], sem_a.at[b]).wait()
            pltpu.make_async_copy(rcv_hbm.at[ridx_all.at[pl.ds(off, K)]],
                                  buf_b.at[b], sem_b.at[b]).wait()

        def fire_write(t, b):
            base = e0 + t * K
            pltpu.async_copy(buf_a.at[b, :, pl.ds(0, 2 * H)],
                             s_out.at[pl.ds(base, K)], sem_w.at[b])
            pltpu.async_copy(dbuf.at[b], d_out.at[pl.ds(base, K)], sem_w.at[b])

        def wait_write(t, b):
            base = e0 + t * K
            pltpu.make_async_copy(buf_a.at[b, :, pl.ds(0, 2 * H)],
                                  s_out.at[pl.ds(base, K)], sem_w.at[b]).wait()
            pltpu.make_async_copy(dbuf.at[b], d_out.at[pl.ds(base, K)],
                                  sem_w.at[b]).wait()

        fire_gather(0, 0)

        def body(t, carry):
            b = t % 2
            nb = 1 - b

            @pl.when(t >= 1)
            def _():
                wait_write(t - 1, nb)

            @pl.when(t + 1 < ch)
            def _():
                fire_gather(t + 1, nb)

            wait_gather(t, b)

            def add_row(i, c):
                for j in range(2 * H // 16):
                    sl = pl.ds(j * 16, 16)
                    buf_a[b, i, sl] = buf_a[b, i, sl] + buf_b[b, i, sl]
                psl = pl.ds(2 * H, 16)
                dbuf[b, i, pl.ds(0, 16)] = buf_a[b, i, psl] + buf_b[b, i, psl]
                return c
            lax.fori_loop(0, K, add_row, 0)
            fire_write(t, b)
            return carry

        lax.fori_loop(0, ch, body, 0)
        wait_write(ch - 1, (ch - 1) % 2)

    return edge_gather


@functools.cache
def _make_edge_scatter(ep):
    epw = ep // NW
    ng = epw // G

    @functools.partial(
        pl.kernel,
        mesh=_mesh,
        out_type=(
            jax.ShapeDtypeStruct((NC, N, H), jnp.float32),
            jax.ShapeDtypeStruct((NC, N, H), jnp.float32),
        ),
        scratch_types=(
            pltpu.VMEM((epw // K2, 1, K2), jnp.int32),
            pltpu.VMEM((2, G, H), jnp.float32),
            pltpu.VMEM_SHARED((N, H), jnp.float32),
            pltpu.SemaphoreType.DMA((2,)),
        ),
    )
    def edge_scatter(msg_hbm, pos_hbm, rec3_hbm, zeros_hbm,
                     out_m, out_p, ridx3, buf, acc, sem_r):
        c = lax.axis_index("c")
        s = lax.axis_index("s")
        wid = s * NC + c
        row0 = s * RPS
        is_last = s == NS - 1
        pltpu.sync_copy(rec3_hbm.at[pl.ds(wid * (epw // K2), epw // K2)],
                        ridx3)

        def scatter_phase(src_hbm, dst_hbm):
            pltpu.sync_copy(zeros_hbm.at[pl.ds(row0, RPS)],
                            acc.at[pl.ds(row0, RPS)])

            @pl.when(is_last)
            def _():
                pltpu.sync_copy(zeros_hbm.at[pl.ds(NS * RPS, NTAIL)],
                                acc.at[pl.ds(NS * RPS, NTAIL)])
            plsc.subcore_barrier()

            def fire_read(r, b):
                base = wid * epw + r * G
                pltpu.async_copy(src_hbm.at[pl.ds(base, G)], buf.at[b],
                                 sem_r.at[b])

            def wait_read(r, b):
                base = wid * epw + r * G
                pltpu.make_async_copy(src_hbm.at[pl.ds(base, G)], buf.at[b],
                                      sem_r.at[b]).wait()

            fire_read(0, 0)

            def chunk(r, carry):
                b = r % 2

                @pl.when(r + 1 < ng)
                def _():
                    fire_read(r + 1, 1 - b)

                wait_read(r, b)
                pltpu.sync_copy(buf.at[b],
                                acc.at[ridx3.at[r, 0]], add=True)
                return carry
            lax.fori_loop(0, ng, chunk, 0)
            plsc.subcore_barrier()
            pltpu.sync_copy(acc.at[pl.ds(row0, RPS)],
                            dst_hbm.at[c, pl.ds(row0, RPS)])

            @pl.when(is_last)
            def _():
                pltpu.sync_copy(acc.at[pl.ds(NS * RPS, NTAIL)],
                                dst_hbm.at[c, pl.ds(NS * RPS, NTAIL)])
            plsc.subcore_barrier()

        scatter_phase(msg_hbm, out_m)
        scatter_phase(pos_hbm, out_p)

    return edge_scatter


def _node_pre_body(x_ref, pe_ref, ppad_ref, wx_ref, wp_ref, b_ref,
                   snd_ref, rcv_ref, ef_ref):
    x = x_ref[:]
    pe = pe_ref[:]
    snd_ref[:, 0:2 * H] = (x @ wx_ref[:, 0:2 * H] + pe @ wp_ref[:, 0:2 * H]
                           + b_ref[:, 0:2 * H])
    snd_ref[:, 2 * H:W] = ppad_ref[:]
    rcv_ref[:, 0:2 * H] = (x @ wx_ref[:, 2 * H:4 * H]
                           + pe @ wp_ref[:, 2 * H:4 * H])
    rcv_ref[:, 2 * H:W] = -ppad_ref[:]
    ef_ref[:] = (x @ wx_ref[:, 4 * H:6 * H] + pe @ wp_ref[:, 4 * H:6 * H]
                 + b_ref[:, 2 * H:4 * H])


def _edge_mlp_body(s_ref, d_ref, wrow_ref, brow_ref, w2_ref, p2_ref,
                   msg_ref, pmsg_ref):
    dvec = d_ref[:]
    dist = jnp.sqrt(jnp.sum(dvec * dvec, axis=1, keepdims=True))
    z1 = s_ref[:, 0:H] + dist * wrow_ref[0:1, :]
    m1 = z1 * jax.nn.sigmoid(z1)
    mm = jnp.dot(m1, w2_ref[:], preferred_element_type=jnp.float32) \
        + brow_ref[0:1, :]
    msg_ref[:] = mm * jax.nn.sigmoid(mm)
    zp = s_ref[:, H:2 * H] + dist * wrow_ref[1:2, :]
    p1 = jnp.tanh(zp)
    pp = jnp.dot(p1, p2_ref[:], preferred_element_type=jnp.float32) \
        + brow_ref[1:2, :]
    pmsg_ref[:] = jnp.tanh(pp)


def _update_body(ef_ref, pm1_ref, pm2_ref, pp1_ref, pp2_ref,
                 u1c_ref, u2_ref, ub2_ref, q1b_ref, q2_ref, qb2_ref,
                 upd_ref, updpe_ref):
    aggr = pm1_ref[0] + pm1_ref[1] + pm2_ref[0] + pm2_ref[1]
    u = ef_ref[:, 0:H] + jnp.dot(aggr, u1c_ref[:],
                                 preferred_element_type=jnp.float32)
    u = u * jax.nn.sigmoid(u)
    upd_ref[:] = jnp.dot(u, u2_ref[:],
                         preferred_element_type=jnp.float32) + ub2_ref[:]
    pos_aggr = pp1_ref[0] + pp1_ref[1] + pp2_ref[0] + pp2_ref[1]
    q = jnp.tanh(ef_ref[:, H:2 * H] + jnp.dot(pos_aggr, q1b_ref[:],
                                              preferred_element_type=jnp.float32))
    updpe_ref[:] = jnp.tanh(jnp.dot(q, q2_ref[:],
                                    preferred_element_type=jnp.float32)
                            + qb2_ref[:])


def kernel(x, pos, pe, edge_index, W1, b1, W2, b2, P1, pb1, P2, pb2,
           U1, ub1, U2, ub2, Q1, qb1, Q2, qb2):
    f32 = jnp.float32
    send = edge_index[0].astype(jnp.int32)
    rec = edge_index[1].astype(jnp.int32)
    ppad = jnp.concatenate([pos.astype(f32),
                            jnp.zeros((N, H - 3), f32)], axis=1)

    zH = jnp.zeros((H, H), f32)
    Wx = jnp.concatenate(
        [W1[0:H], zH, W1[2 * H:3 * H], zH, U1[0:H], zH], axis=1)
    Wp = jnp.concatenate(
        [W1[H:2 * H], P1[0:H], W1[3 * H:4 * H], P1[H:2 * H],
         U1[H:2 * H], Q1[0:H]], axis=1)
    bias = jnp.concatenate(
        [b1, pb1, ub1, qb1]).reshape(1, 4 * H)

    Tn = 2000
    snd_t, rcv_t, ef_t = pl.pallas_call(
        _node_pre_body,
        grid=(N // Tn,),
        in_specs=[
            pl.BlockSpec((Tn, H), lambda i: (i, 0)),
            pl.BlockSpec((Tn, H), lambda i: (i, 0)),
            pl.BlockSpec((Tn, H), lambda i: (i, 0)),
            pl.BlockSpec((H, 6 * H), lambda i: (0, 0)),
            pl.BlockSpec((H, 6 * H), lambda i: (0, 0)),
            pl.BlockSpec((1, 4 * H), lambda i: (0, 0)),
        ],
        out_specs=[
            pl.BlockSpec((Tn, W), lambda i: (i, 0)),
            pl.BlockSpec((Tn, W), lambda i: (i, 0)),
            pl.BlockSpec((Tn, 2 * H), lambda i: (i, 0)),
        ],
        out_shape=[
            jax.ShapeDtypeStruct((N, W), f32),
            jax.ShapeDtypeStruct((N, W), f32),
            jax.ShapeDtypeStruct((N, 2 * H), f32),
        ],
    )(x, pe, ppad, Wx, Wp, bias)

    wrow = jnp.stack([W1[4 * H], P1[2 * H]])
    brow = jnp.stack([b2, pb2])

    P = 2
    Eh = E // P
    gather_fn = _make_edge_gather(Eh)
    scatter_fn = _make_edge_scatter(Eh)
    Te = 2000
    zeros_nh = jnp.zeros((N, H), f32)

    partials = []
    for p in range(P):
        sl = slice(p * Eh, (p + 1) * Eh)
        s_edge, d_edge = gather_fn(snd_t, rcv_t, send[sl], rec[sl])
        msg, pmsg = pl.pallas_call(
            _edge_mlp_body,
            grid=(Eh // Te,),
            in_specs=[
                pl.BlockSpec((Te, 2 * H), lambda i: (i, 0)),
                pl.BlockSpec((Te, 16), lambda i: (i, 0)),
                pl.BlockSpec((2, H), lambda i: (0, 0)),
                pl.BlockSpec((2, H), lambda i: (0, 0)),
                pl.BlockSpec((H, H), lambda i: (0, 0)),
                pl.BlockSpec((H, H), lambda i: (0, 0)),
            ],
            out_specs=[
                pl.BlockSpec((Te, H), lambda i: (i, 0)),
                pl.BlockSpec((Te, H), lambda i: (i, 0)),
            ],
            out_shape=[
                jax.ShapeDtypeStruct((Eh, H), f32),
                jax.ShapeDtypeStruct((Eh, H), f32),
            ],
        )(s_edge, d_edge, wrow, brow, W2, P2)
        rec3 = rec[sl].reshape(Eh // K2, 1, K2)
        pm, pp = scatter_fn(msg, pmsg, rec3, zeros_nh)
        partials.append((pm, pp))

    (pm1, pp1), (pm2, pp2) = partials
    upd, upd_pe = pl.pallas_call(
        _update_body,
        grid=(N // Tn,),
        in_specs=[
            pl.BlockSpec((Tn, 2 * H), lambda i: (i, 0)),
            pl.BlockSpec((NC, Tn, H), lambda i: (0, i, 0)),
            pl.BlockSpec((NC, Tn, H), lambda i: (0, i, 0)),
            pl.BlockSpec((NC, Tn, H), lambda i: (0, i, 0)),
            pl.BlockSpec((NC, Tn, H), lambda i: (0, i, 0)),
            pl.BlockSpec((H, H), lambda i: (0, 0)),
            pl.BlockSpec((H, H), lambda i: (0, 0)),
            pl.BlockSpec((1, H), lambda i: (0, 0)),
            pl.BlockSpec((H, H), lambda i: (0, 0)),
            pl.BlockSpec((H, H), lambda i: (0, 0)),
            pl.BlockSpec((1, H), lambda i: (0, 0)),
        ],
        out_specs=[
            pl.BlockSpec((Tn, H), lambda i: (i, 0)),
            pl.BlockSpec((Tn, H), lambda i: (i, 0)),
        ],
        out_shape=[
            jax.ShapeDtypeStruct((N, H), f32),
            jax.ShapeDtypeStruct((N, H), f32),
        ],
    )(ef_t, pm1, pm2, pp1, pp2, U1[2 * H:3 * H], U2, ub2.reshape(1, H),
      Q1[H:2 * H], Q2, qb2.reshape(1, H))

    return (upd, upd_pe)


# speedup vs baseline: 1.0639x; 1.0639x over previous
import functools

import jax
import jax.numpy as jnp
from jax import lax
from jax.experimental import pallas as pl
from jax.experimental.pallas import tpu as pltpu
from jax.experimental.pallas import tpu_sc as plsc

N = 10000
E = 320000
H = 128
W = 3 * H

NC = 2
NS = 16
NW = NC * NS
K = 40
K2 = 40
G = 80
RPS = 624
NTAIL = N - NS * RPS

_mesh = plsc.VectorSubcoreMesh(core_axis_name="c", subcore_axis_name="s")


@functools.cache
def _make_edge_gather(ep):
    epw = ep // NW
    ch = epw // K

    @functools.partial(
        pl.kernel,
        mesh=_mesh,
        out_type=(
            jax.ShapeDtypeStruct((ep, 2 * H), jnp.float32),
            jax.ShapeDtypeStruct((ep, 16), jnp.float32),
        ),
        scratch_types=(
            pltpu.VMEM((epw,), jnp.int32),
            pltpu.VMEM((epw,), jnp.int32),
            pltpu.VMEM((2, K, W), jnp.float32),
            pltpu.VMEM((2, K, W), jnp.float32),
            pltpu.VMEM((2, K, 16), jnp.float32),
            pltpu.SemaphoreType.DMA((2,)),
            pltpu.SemaphoreType.DMA((2,)),
            pltpu.SemaphoreType.DMA((2,)),
        ),
    )
    def edge_gather(snd_hbm, rcv_hbm, send_hbm, rec_hbm, s_out, d_out,
                    sidx_all, ridx_all, buf_a, buf_b, dbuf,
                    sem_a, sem_b, sem_w):
        wid = lax.axis_index("s") * NC + lax.axis_index("c")
        e0 = wid * epw
        pltpu.sync_copy(send_hbm.at[pl.ds(e0, epw)], sidx_all)
        pltpu.sync_copy(rec_hbm.at[pl.ds(e0, epw)], ridx_all)

        def fire_gather(t, b):
            off = t * K
            pltpu.async_copy(snd_hbm.at[sidx_all.at[pl.ds(off, K)]],
                             buf_a.at[b], sem_a.at[b])
            pltpu.async_copy(rcv_hbm.at[ridx_all.at[pl.ds(off, K)]],
                             buf_b.at[b], sem_b.at[b])

        def wait_gather(t, b):
            off = t * K
            pltpu.make_async_copy(snd_hbm.at[sidx_all.at[pl.ds(off, K)]],
                                  buf_a.at[b], sem_a.at[b]).wait()
            pltpu.make_async_copy(rcv_hbm.at[ridx_all.at[pl.ds(off, K)]],
                                  buf_b.at[b], sem_b.at[b]).wait()

        def fire_write(t, b):
            base = e0 + t * K
            pltpu.async_copy(buf_a.at[b, :, pl.ds(0, 2 * H)],
                             s_out.at[pl.ds(base, K)], sem_w.at[b])
            pltpu.async_copy(dbuf.at[b], d_out.at[pl.ds(base, K)], sem_w.at[b])

        def wait_write(t, b):
            base = e0 + t * K
            pltpu.make_async_copy(buf_a.at[b, :, pl.ds(0, 2 * H)],
                                  s_out.at[pl.ds(base, K)], sem_w.at[b]).wait()
            pltpu.make_async_copy(dbuf.at[b], d_out.at[pl.ds(base, K)],
                                  sem_w.at[b]).wait()

        fire_gather(0, 0)

        def body(t, carry):
            b = t % 2
            nb = 1 - b

            @pl.when(t >= 1)
            def _():
                wait_write(t - 1, nb)

            @pl.when(t + 1 < ch)
            def _():
                fire_gather(t + 1, nb)

            wait_gather(t, b)

            def add_row(i, c):
                for j in range(2 * H // 16):
                    sl = pl.ds(j * 16, 16)
                    buf_a[b, i, sl] = buf_a[b, i, sl] + buf_b[b, i, sl]
                psl = pl.ds(2 * H, 16)
                dbuf[b, i, pl.ds(0, 16)] = buf_a[b, i, psl] + buf_b[b, i, psl]
                return c
            lax.fori_loop(0, K, add_row, 0)
            fire_write(t, b)
            return carry

        lax.fori_loop(0, ch, body, 0)
        wait_write(ch - 1, (ch - 1) % 2)

    return edge_gather


@functools.cache
def _make_edge_scatter(ep):
    epw = ep // NW
    ng = epw // G
    tail = epw - ng * G

    @functools.partial(
        pl.kernel,
        mesh=_mesh,
        out_type=(
            jax.ShapeDtypeStruct((NC, N, H), jnp.float32),
            jax.ShapeDtypeStruct((NC, N, H), jnp.float32),
        ),
        scratch_types=(
            pltpu.VMEM((epw // K2, 1, K2), jnp.int32),
            pltpu.VMEM((2, G, H), jnp.float32),
            pltpu.VMEM_SHARED((N, H), jnp.float32),
            pltpu.SemaphoreType.DMA((2,)),
        ),
    )
    def edge_scatter(msg_hbm, pos_hbm, rec3_hbm, zeros_hbm,
                     out_m, out_p, ridx3, buf, acc, sem_r):
        c = lax.axis_index("c")
        s = lax.axis_index("s")
        wid = s * NC + c
        row0 = s * RPS
        is_last = s == NS - 1
        pltpu.sync_copy(rec3_hbm.at[pl.ds(wid * (epw // K2), epw // K2)],
                        ridx3)

        def scatter_phase(src_hbm, dst_hbm):
            pltpu.sync_copy(zeros_hbm.at[pl.ds(row0, RPS)],
                            acc.at[pl.ds(row0, RPS)])

            @pl.when(is_last)
            def _():
                pltpu.sync_copy(zeros_hbm.at[pl.ds(NS * RPS, NTAIL)],
                                acc.at[pl.ds(NS * RPS, NTAIL)])
            plsc.subcore_barrier()

            def fire_read(r, b):
                base = wid * epw + r * G
                pltpu.async_copy(src_hbm.at[pl.ds(base, G)], buf.at[b],
                                 sem_r.at[b])

            def wait_read(r, b):
                base = wid * epw + r * G
                pltpu.make_async_copy(src_hbm.at[pl.ds(base, G)], buf.at[b],
                                      sem_r.at[b]).wait()

            fire_read(0, 0)

            def chunk(r, carry):
                b = r % 2

                @pl.when(r + 1 < ng)
                def _():
                    fire_read(r + 1, 1 - b)

                wait_read(r, b)
                for j in range(G // K2):
                    pltpu.sync_copy(buf.at[b, pl.ds(j * K2, K2)],
                                    acc.at[ridx3.at[r * (G // K2) + j, 0]],
                                    add=True)
                return carry
            lax.fori_loop(0, ng, chunk, 0)
            if tail:
                base = wid * epw + ng * G
                pltpu.sync_copy(src_hbm.at[pl.ds(base, tail)],
                                buf.at[0, pl.ds(0, tail)])
                for j in range(tail // K2):
                    pltpu.sync_copy(buf.at[0, pl.ds(j * K2, K2)],
                                    acc.at[ridx3.at[ng * (G // K2) + j, 0]],
                                    add=True)
            plsc.subcore_barrier()
            pltpu.sync_copy(acc.at[pl.ds(row0, RPS)],
                            dst_hbm.at[c, pl.ds(row0, RPS)])

            @pl.when(is_last)
            def _():
                pltpu.sync_copy(acc.at[pl.ds(NS * RPS, NTAIL)],
                                dst_hbm.at[c, pl.ds(NS * RPS, NTAIL)])
            plsc.subcore_barrier()

        scatter_phase(msg_hbm, out_m)
        scatter_phase(pos_hbm, out_p)

    return edge_scatter


def _node_pre_body(x_ref, pe_ref, ppad_ref, wx_ref, wp_ref, b_ref,
                   snd_ref, rcv_ref, ef_ref):
    x = x_ref[:]
    pe = pe_ref[:]
    snd_ref[:, 0:2 * H] = (x @ wx_ref[:, 0:2 * H] + pe @ wp_ref[:, 0:2 * H]
                           + b_ref[:, 0:2 * H])
    snd_ref[:, 2 * H:W] = ppad_ref[:]
    rcv_ref[:, 0:2 * H] = (x @ wx_ref[:, 2 * H:4 * H]
                           + pe @ wp_ref[:, 2 * H:4 * H])
    rcv_ref[:, 2 * H:W] = -ppad_ref[:]
    ef_ref[:] = (x @ wx_ref[:, 4 * H:6 * H] + pe @ wp_ref[:, 4 * H:6 * H]
                 + b_ref[:, 2 * H:4 * H])


def _edge_mlp_body(s_ref, d_ref, wrow_ref, brow_ref, w2_ref, p2_ref,
                   msg_ref, pmsg_ref):
    dvec = d_ref[:]
    dist = jnp.sqrt(jnp.sum(dvec * dvec, axis=1, keepdims=True))
    z1 = s_ref[:, 0:H] + dist * wrow_ref[0:1, :]
    m1 = z1 * jax.nn.sigmoid(z1)
    mm = jnp.dot(m1, w2_ref[:], preferred_element_type=jnp.float32) \
        + brow_ref[0:1, :]
    msg_ref[:] = mm * jax.nn.sigmoid(mm)
    zp = s_ref[:, H:2 * H] + dist * wrow_ref[1:2, :]
    p1 = jnp.tanh(zp)
    pp = jnp.dot(p1, p2_ref[:], preferred_element_type=jnp.float32) \
        + brow_ref[1:2, :]
    pmsg_ref[:] = jnp.tanh(pp)


def _update_body(ef_ref, pm1_ref, pm2_ref, pp1_ref, pp2_ref,
                 u1c_ref, u2_ref, ub2_ref, q1b_ref, q2_ref, qb2_ref,
                 upd_ref, updpe_ref):
    aggr = pm1_ref[0] + pm1_ref[1] + pm2_ref[0] + pm2_ref[1]
    u = ef_ref[:, 0:H] + jnp.dot(aggr, u1c_ref[:],
                                 preferred_element_type=jnp.float32)
    u = u * jax.nn.sigmoid(u)
    upd_ref[:] = jnp.dot(u, u2_ref[:],
                         preferred_element_type=jnp.float32) + ub2_ref[:]
    pos_aggr = pp1_ref[0] + pp1_ref[1] + pp2_ref[0] + pp2_ref[1]
    q = jnp.tanh(ef_ref[:, H:2 * H] + jnp.dot(pos_aggr, q1b_ref[:],
                                              preferred_element_type=jnp.float32))
    updpe_ref[:] = jnp.tanh(jnp.dot(q, q2_ref[:],
                                    preferred_element_type=jnp.float32)
                            + qb2_ref[:])


def kernel(x, pos, pe, edge_index, W1, b1, W2, b2, P1, pb1, P2, pb2,
           U1, ub1, U2, ub2, Q1, qb1, Q2, qb2):
    f32 = jnp.float32
    send = edge_index[0].astype(jnp.int32)
    rec = edge_index[1].astype(jnp.int32)
    ppad = jnp.concatenate([pos.astype(f32),
                            jnp.zeros((N, H - 3), f32)], axis=1)

    zH = jnp.zeros((H, H), f32)
    Wx = jnp.concatenate(
        [W1[0:H], zH, W1[2 * H:3 * H], zH, U1[0:H], zH], axis=1)
    Wp = jnp.concatenate(
        [W1[H:2 * H], P1[0:H], W1[3 * H:4 * H], P1[H:2 * H],
         U1[H:2 * H], Q1[0:H]], axis=1)
    bias = jnp.concatenate(
        [b1, pb1, ub1, qb1]).reshape(1, 4 * H)

    Tn = 2000
    snd_t, rcv_t, ef_t = pl.pallas_call(
        _node_pre_body,
        grid=(N // Tn,),
        in_specs=[
            pl.BlockSpec((Tn, H), lambda i: (i, 0)),
            pl.BlockSpec((Tn, H), lambda i: (i, 0)),
            pl.BlockSpec((Tn, H), lambda i: (i, 0)),
            pl.BlockSpec((H, 6 * H), lambda i: (0, 0)),
            pl.BlockSpec((H, 6 * H), lambda i: (0, 0)),
            pl.BlockSpec((1, 4 * H), lambda i: (0, 0)),
        ],
        out_specs=[
            pl.BlockSpec((Tn, W), lambda i: (i, 0)),
            pl.BlockSpec((Tn, W), lambda i: (i, 0)),
            pl.BlockSpec((Tn, 2 * H), lambda i: (i, 0)),
        ],
        out_shape=[
            jax.ShapeDtypeStruct((N, W), f32),
            jax.ShapeDtypeStruct((N, W), f32),
            jax.ShapeDtypeStruct((N, 2 * H), f32),
        ],
    )(x, pe, ppad, Wx, Wp, bias)

    wrow = jnp.stack([W1[4 * H], P1[2 * H]])
    brow = jnp.stack([b2, pb2])

    P = 2
    Eh = E // P
    gather_fn = _make_edge_gather(Eh)
    scatter_fn = _make_edge_scatter(Eh)
    Te = 2000
    zeros_nh = jnp.zeros((N, H), f32)

    partials = []
    for p in range(P):
        sl = slice(p * Eh, (p + 1) * Eh)
        s_edge, d_edge = gather_fn(snd_t, rcv_t, send[sl], rec[sl])
        msg, pmsg = pl.pallas_call(
            _edge_mlp_body,
            grid=(Eh // Te,),
            in_specs=[
                pl.BlockSpec((Te, 2 * H), lambda i: (i, 0)),
                pl.BlockSpec((Te, 16), lambda i: (i, 0)),
                pl.BlockSpec((2, H), lambda i: (0, 0)),
                pl.BlockSpec((2, H), lambda i: (0, 0)),
                pl.BlockSpec((H, H), lambda i: (0, 0)),
                pl.BlockSpec((H, H), lambda i: (0, 0)),
            ],
            out_specs=[
                pl.BlockSpec((Te, H), lambda i: (i, 0)),
                pl.BlockSpec((Te, H), lambda i: (i, 0)),
            ],
            out_shape=[
                jax.ShapeDtypeStruct((Eh, H), f32),
                jax.ShapeDtypeStruct((Eh, H), f32),
            ],
        )(s_edge, d_edge, wrow, brow, W2, P2)
        rec3 = rec[sl].reshape(Eh // K2, 1, K2)
        pm, pp = scatter_fn(msg, pmsg, rec3, zeros_nh)
        partials.append((pm, pp))

    (pm1, pp1), (pm2, pp2) = partials
    upd, upd_pe = pl.pallas_call(
        _update_body,
        grid=(N // Tn,),
        in_specs=[
            pl.BlockSpec((Tn, 2 * H), lambda i: (i, 0)),
            pl.BlockSpec((NC, Tn, H), lambda i: (0, i, 0)),
            pl.BlockSpec((NC, Tn, H), lambda i: (0, i, 0)),
            pl.BlockSpec((NC, Tn, H), lambda i: (0, i, 0)),
            pl.BlockSpec((NC, Tn, H), lambda i: (0, i, 0)),
            pl.BlockSpec((H, H), lambda i: (0, 0)),
            pl.BlockSpec((H, H), lambda i: (0, 0)),
            pl.BlockSpec((1, H), lambda i: (0, 0)),
            pl.BlockSpec((H, H), lambda i: (0, 0)),
            pl.BlockSpec((H, H), lambda i: (0, 0)),
            pl.BlockSpec((1, H), lambda i: (0, 0)),
        ],
        out_specs=[
            pl.BlockSpec((Tn, H), lambda i: (i, 0)),
            pl.BlockSpec((Tn, H), lambda i: (i, 0)),
        ],
        out_shape=[
            jax.ShapeDtypeStruct((N, H), f32),
            jax.ShapeDtypeStruct((N, H), f32),
        ],
    )(ef_t, pm1, pm2, pp1, pp2, U1[2 * H:3 * H], U2, ub2.reshape(1, H),
      Q1[H:2 * H], Q2, qb2.reshape(1, H))

    return (upd, upd_pe)


# speedup vs baseline: 1.0667x; 1.0027x over previous
import functools

import jax
import jax.numpy as jnp
from jax import lax
from jax.experimental import pallas as pl
from jax.experimental.pallas import tpu as pltpu
from jax.experimental.pallas import tpu_sc as plsc

N = 10000
E = 320000
H = 128
W = 3 * H

NC = 2
NS = 16
NW = NC * NS
K = 40
K2 = 40
G = 80
RPS = 624
NTAIL = N - NS * RPS

_mesh = plsc.VectorSubcoreMesh(core_axis_name="c", subcore_axis_name="s")


@functools.cache
def _make_edge_gather(ep):
    epw = ep // NW
    ch = epw // K

    @functools.partial(
        pl.kernel,
        mesh=_mesh,
        out_type=(
            jax.ShapeDtypeStruct((ep, 2 * H), jnp.float32),
            jax.ShapeDtypeStruct((ep, 16), jnp.float32),
        ),
        scratch_types=(
            pltpu.VMEM((epw,), jnp.int32),
            pltpu.VMEM((epw,), jnp.int32),
            pltpu.VMEM((3, K, W), jnp.float32),
            pltpu.VMEM((3, K, W), jnp.float32),
            pltpu.VMEM((3, K, 16), jnp.float32),
            pltpu.SemaphoreType.DMA((3,)),
            pltpu.SemaphoreType.DMA((3,)),
            pltpu.SemaphoreType.DMA((3,)),
        ),
    )
    def edge_gather(snd_hbm, rcv_hbm, send_hbm, rec_hbm, s_out, d_out,
                    sidx_all, ridx_all, buf_a, buf_b, dbuf,
                    sem_a, sem_b, sem_w):
        wid = lax.axis_index("s") * NC + lax.axis_index("c")
        e0 = wid * epw
        pltpu.sync_copy(send_hbm.at[pl.ds(e0, epw)], sidx_all)
        pltpu.sync_copy(rec_hbm.at[pl.ds(e0, epw)], ridx_all)

        def fire_gather(t, b):
            off = t * K
            pltpu.async_copy(snd_hbm.at[sidx_all.at[pl.ds(off, K)]],
                             buf_a.at[b], sem_a.at[b])
            pltpu.async_copy(rcv_hbm.at[ridx_all.at[pl.ds(off, K)]],
                             buf_b.at[b], sem_b.at[b])

        def wait_gather(t, b):
            off = t * K
            pltpu.make_async_copy(snd_hbm.at[sidx_all.at[pl.ds(off, K)]],
                                  buf_a.at[b], sem_a.at[b]).wait()
            pltpu.make_async_copy(rcv_hbm.at[ridx_all.at[pl.ds(off, K)]],
                                  buf_b.at[b], sem_b.at[b]).wait()

        def fire_write(t, b):
            base = e0 + t * K
            pltpu.async_copy(buf_a.at[b, :, pl.ds(0, 2 * H)],
                             s_out.at[pl.ds(base, K)], sem_w.at[b])
            pltpu.async_copy(dbuf.at[b], d_out.at[pl.ds(base, K)], sem_w.at[b])

        def wait_write(t, b):
            base = e0 + t * K
            pltpu.make_async_copy(buf_a.at[b, :, pl.ds(0, 2 * H)],
                                  s_out.at[pl.ds(base, K)], sem_w.at[b]).wait()
            pltpu.make_async_copy(dbuf.at[b], d_out.at[pl.ds(base, K)],
                                  sem_w.at[b]).wait()

        fire_gather(0, 0)
        fire_gather(1, 1)

        def body(t, carry):
            b = t % 3
            fb = (t + 2) % 3

            @pl.when(t >= 1)
            def _():
                wait_write(t - 1, fb)

            @pl.when(t + 2 < ch)
            def _():
                fire_gather(t + 2, fb)

            wait_gather(t, b)

            def add_row(i, c):
                for j in range(2 * H // 16):
                    sl = pl.ds(j * 16, 16)
                    buf_a[b, i, sl] = buf_a[b, i, sl] + buf_b[b, i, sl]
                psl = pl.ds(2 * H, 16)
                dbuf[b, i, pl.ds(0, 16)] = buf_a[b, i, psl] + buf_b[b, i, psl]
                return c
            lax.fori_loop(0, K, add_row, 0)
            fire_write(t, b)
            return carry

        lax.fori_loop(0, ch, body, 0)
        wait_write(ch - 1, (ch - 1) % 3)

    return edge_gather


@functools.cache
def _make_edge_scatter(ep):
    epw = ep // NW
    ng = epw // G
    tail = epw - ng * G

    @functools.partial(
        pl.kernel,
        mesh=_mesh,
        out_type=(
            jax.ShapeDtypeStruct((NC, N, H), jnp.float32),
            jax.ShapeDtypeStruct((NC, N, H), jnp.float32),
        ),
        scratch_types=(
            pltpu.VMEM((epw // K2, 1, K2), jnp.int32),
            pltpu.VMEM((2, G, H), jnp.float32),
            pltpu.VMEM_SHARED((N, H), jnp.float32),
            pltpu.SemaphoreType.DMA((2,)),
        ),
    )
    def edge_scatter(msg_hbm, pos_hbm, rec3_hbm, zeros_hbm,
                     out_m, out_p, ridx3, buf, acc, sem_r):
        c = lax.axis_index("c")
        s = lax.axis_index("s")
        wid = s * NC + c
        row0 = s * RPS
        is_last = s == NS - 1
        pltpu.sync_copy(rec3_hbm.at[pl.ds(wid * (epw // K2), epw // K2)],
                        ridx3)

        def scatter_phase(src_hbm, dst_hbm):
            pltpu.sync_copy(zeros_hbm.at[pl.ds(row0, RPS)],
                            acc.at[pl.ds(row0, RPS)])

            @pl.when(is_last)
            def _():
                pltpu.sync_copy(zeros_hbm.at[pl.ds(NS * RPS, NTAIL)],
                                acc.at[pl.ds(NS * RPS, NTAIL)])
            plsc.subcore_barrier()

            def fire_read(r, b):
                base = wid * epw + r * G
                pltpu.async_copy(src_hbm.at[pl.ds(base, G)], buf.at[b],
                                 sem_r.at[b])

            def wait_read(r, b):
                base = wid * epw + r * G
                pltpu.make_async_copy(src_hbm.at[pl.ds(base, G)], buf.at[b],
                                      sem_r.at[b]).wait()

            fire_read(0, 0)

            def chunk(r, carry):
                b = r % 2

                @pl.when(r + 1 < ng)
                def _():
                    fire_read(r + 1, 1 - b)

                wait_read(r, b)
                for j in range(G // K2):
                    pltpu.sync_copy(buf.at[b, pl.ds(j * K2, K2)],
                                    acc.at[ridx3.at[r * (G // K2) + j, 0]],
                                    add=True)
                return carry
            lax.fori_loop(0, ng, chunk, 0)
            if tail:
                base = wid * epw + ng * G
                pltpu.sync_copy(src_hbm.at[pl.ds(base, tail)],
                                buf.at[0, pl.ds(0, tail)])
                for j in range(tail // K2):
                    pltpu.sync_copy(buf.at[0, pl.ds(j * K2, K2)],
                                    acc.at[ridx3.at[ng * (G // K2) + j, 0]],
                                    add=True)
            plsc.subcore_barrier()
            pltpu.sync_copy(acc.at[pl.ds(row0, RPS)],
                            dst_hbm.at[c, pl.ds(row0, RPS)])

            @pl.when(is_last)
            def _():
                pltpu.sync_copy(acc.at[pl.ds(NS * RPS, NTAIL)],
                                dst_hbm.at[c, pl.ds(NS * RPS, NTAIL)])
            plsc.subcore_barrier()

        scatter_phase(msg_hbm, out_m)
        scatter_phase(pos_hbm, out_p)

    return edge_scatter


def _node_pre_body(x_ref, pe_ref, ppad_ref, wx_ref, wp_ref, b_ref,
                   snd_ref, rcv_ref, ef_ref):
    x = x_ref[:]
    pe = pe_ref[:]
    snd_ref[:, 0:2 * H] = (x @ wx_ref[:, 0:2 * H] + pe @ wp_ref[:, 0:2 * H]
                           + b_ref[:, 0:2 * H])
    snd_ref[:, 2 * H:W] = ppad_ref[:]
    rcv_ref[:, 0:2 * H] = (x @ wx_ref[:, 2 * H:4 * H]
                           + pe @ wp_ref[:, 2 * H:4 * H])
    rcv_ref[:, 2 * H:W] = -ppad_ref[:]
    ef_ref[:] = (x @ wx_ref[:, 4 * H:6 * H] + pe @ wp_ref[:, 4 * H:6 * H]
                 + b_ref[:, 2 * H:4 * H])


def _edge_mlp_body(s_ref, d_ref, wrow_ref, brow_ref, w2_ref, p2_ref,
                   msg_ref, pmsg_ref):
    dvec = d_ref[:]
    dist = jnp.sqrt(jnp.sum(dvec * dvec, axis=1, keepdims=True))
    z1 = s_ref[:, 0:H] + dist * wrow_ref[0:1, :]
    m1 = z1 * jax.nn.sigmoid(z1)
    mm = jnp.dot(m1, w2_ref[:], preferred_element_type=jnp.float32) \
        + brow_ref[0:1, :]
    msg_ref[:] = mm * jax.nn.sigmoid(mm)
    zp = s_ref[:, H:2 * H] + dist * wrow_ref[1:2, :]
    p1 = jnp.tanh(zp)
    pp = jnp.dot(p1, p2_ref[:], preferred_element_type=jnp.float32) \
        + brow_ref[1:2, :]
    pmsg_ref[:] = jnp.tanh(pp)


def _update_body(ef_ref, pm1_ref, pm2_ref, pp1_ref, pp2_ref,
                 u1c_ref, u2_ref, ub2_ref, q1b_ref, q2_ref, qb2_ref,
                 upd_ref, updpe_ref):
    aggr = pm1_ref[0] + pm1_ref[1] + pm2_ref[0] + pm2_ref[1]
    u = ef_ref[:, 0:H] + jnp.dot(aggr, u1c_ref[:],
                                 preferred_element_type=jnp.float32)
    u = u * jax.nn.sigmoid(u)
    upd_ref[:] = jnp.dot(u, u2_ref[:],
                         preferred_element_type=jnp.float32) + ub2_ref[:]
    pos_aggr = pp1_ref[0] + pp1_ref[1] + pp2_ref[0] + pp2_ref[1]
    q = jnp.tanh(ef_ref[:, H:2 * H] + jnp.dot(pos_aggr, q1b_ref[:],
                                              preferred_element_type=jnp.float32))
    updpe_ref[:] = jnp.tanh(jnp.dot(q, q2_ref[:],
                                    preferred_element_type=jnp.float32)
                            + qb2_ref[:])


def kernel(x, pos, pe, edge_index, W1, b1, W2, b2, P1, pb1, P2, pb2,
           U1, ub1, U2, ub2, Q1, qb1, Q2, qb2):
    f32 = jnp.float32
    send = edge_index[0].astype(jnp.int32)
    rec = edge_index[1].astype(jnp.int32)
    ppad = jnp.concatenate([pos.astype(f32),
                            jnp.zeros((N, H - 3), f32)], axis=1)

    zH = jnp.zeros((H, H), f32)
    Wx = jnp.concatenate(
        [W1[0:H], zH, W1[2 * H:3 * H], zH, U1[0:H], zH], axis=1)
    Wp = jnp.concatenate(
        [W1[H:2 * H], P1[0:H], W1[3 * H:4 * H], P1[H:2 * H],
         U1[H:2 * H], Q1[0:H]], axis=1)
    bias = jnp.concatenate(
        [b1, pb1, ub1, qb1]).reshape(1, 4 * H)

    Tn = 2000
    snd_t, rcv_t, ef_t = pl.pallas_call(
        _node_pre_body,
        grid=(N // Tn,),
        in_specs=[
            pl.BlockSpec((Tn, H), lambda i: (i, 0)),
            pl.BlockSpec((Tn, H), lambda i: (i, 0)),
            pl.BlockSpec((Tn, H), lambda i: (i, 0)),
            pl.BlockSpec((H, 6 * H), lambda i: (0, 0)),
            pl.BlockSpec((H, 6 * H), lambda i: (0, 0)),
            pl.BlockSpec((1, 4 * H), lambda i: (0, 0)),
        ],
        out_specs=[
            pl.BlockSpec((Tn, W), lambda i: (i, 0)),
            pl.BlockSpec((Tn, W), lambda i: (i, 0)),
            pl.BlockSpec((Tn, 2 * H), lambda i: (i, 0)),
        ],
        out_shape=[
            jax.ShapeDtypeStruct((N, W), f32),
            jax.ShapeDtypeStruct((N, W), f32),
            jax.ShapeDtypeStruct((N, 2 * H), f32),
        ],
    )(x, pe, ppad, Wx, Wp, bias)

    wrow = jnp.stack([W1[4 * H], P1[2 * H]])
    brow = jnp.stack([b2, pb2])

    P = 2
    Eh = E // P
    gather_fn = _make_edge_gather(Eh)
    scatter_fn = _make_edge_scatter(Eh)
    Te = 2000
    zeros_nh = jnp.zeros((N, H), f32)

    partials = []
    for p in range(P):
        sl = slice(p * Eh, (p + 1) * Eh)
        s_edge, d_edge = gather_fn(snd_t, rcv_t, send[sl], rec[sl])
        msg, pmsg = pl.pallas_call(
            _edge_mlp_body,
            grid=(Eh // Te,),
            in_specs=[
                pl.BlockSpec((Te, 2 * H), lambda i: (i, 0)),
                pl.BlockSpec((Te, 16), lambda i: (i, 0)),
                pl.BlockSpec((2, H), lambda i: (0, 0)),
                pl.BlockSpec((2, H), lambda i: (0, 0)),
                pl.BlockSpec((H, H), lambda i: (0, 0)),
                pl.BlockSpec((H, H), lambda i: (0, 0)),
            ],
            out_specs=[
                pl.BlockSpec((Te, H), lambda i: (i, 0)),
                pl.BlockSpec((Te, H), lambda i: (i, 0)),
            ],
            out_shape=[
                jax.ShapeDtypeStruct((Eh, H), f32),
                jax.ShapeDtypeStruct((Eh, H), f32),
            ],
        )(s_edge, d_edge, wrow, brow, W2, P2)
        rec3 = rec[sl].reshape(Eh // K2, 1, K2)
        pm, pp = scatter_fn(msg, pmsg, rec3, zeros_nh)
        partials.append((pm, pp))

    (pm1, pp1), (pm2, pp2) = partials
    upd, upd_pe = pl.pallas_call(
        _update_body,
        grid=(N // Tn,),
        in_specs=[
            pl.BlockSpec((Tn, 2 * H), lambda i: (i, 0)),
            pl.BlockSpec((NC, Tn, H), lambda i: (0, i, 0)),
            pl.BlockSpec((NC, Tn, H), lambda i: (0, i, 0)),
            pl.BlockSpec((NC, Tn, H), lambda i: (0, i, 0)),
            pl.BlockSpec((NC, Tn, H), lambda i: (0, i, 0)),
            pl.BlockSpec((H, H), lambda i: (0, 0)),
            pl.BlockSpec((H, H), lambda i: (0, 0)),
            pl.BlockSpec((1, H), lambda i: (0, 0)),
            pl.BlockSpec((H, H), lambda i: (0, 0)),
            pl.BlockSpec((H, H), lambda i: (0, 0)),
            pl.BlockSpec((1, H), lambda i: (0, 0)),
        ],
        out_specs=[
            pl.BlockSpec((Tn, H), lambda i: (i, 0)),
            pl.BlockSpec((Tn, H), lambda i: (i, 0)),
        ],
        out_shape=[
            jax.ShapeDtypeStruct((N, H), f32),
            jax.ShapeDtypeStruct((N, H), f32),
        ],
    )(ef_t, pm1, pm2, pp1, pp2, U1[2 * H:3 * H], U2, ub2.reshape(1, H),
      Q1[H:2 * H], Q2, qb2.reshape(1, H))

    return (upd, upd_pe)


# speedup vs baseline: 1.1963x; 1.1215x over previous
import functools

import jax
import jax.numpy as jnp
from jax import lax
from jax.experimental import pallas as pl
from jax.experimental.pallas import tpu as pltpu
from jax.experimental.pallas import tpu_sc as plsc

N = 10000
E = 320000
H = 128
W = 3 * H

NC = 2
NS = 16
NW = NC * NS
K = 40
K2 = 40
G = 80
RPS = 624
NTAIL = N - NS * RPS

_mesh = plsc.VectorSubcoreMesh(core_axis_name="c", subcore_axis_name="s")


@functools.cache
def _make_edge_gather(ep):
    epw = ep // NW
    ch = epw // K

    @functools.partial(
        pl.kernel,
        mesh=_mesh,
        out_type=(
            jax.ShapeDtypeStruct((ep, H), jnp.float32),
            jax.ShapeDtypeStruct((ep, H), jnp.float32),
            jax.ShapeDtypeStruct((ep, 16), jnp.float32),
        ),
        scratch_types=(
            pltpu.VMEM((epw,), jnp.int32),
            pltpu.VMEM((epw,), jnp.int32),
            pltpu.VMEM((3, K, 2 * H), jnp.float32),
            pltpu.VMEM((3, K, 2 * H), jnp.float32),
            pltpu.VMEM((3, K, 16), jnp.float32),
            pltpu.SemaphoreType.DMA((3,)),
            pltpu.SemaphoreType.DMA((3,)),
            pltpu.SemaphoreType.DMA((3,)),
        ),
    )
    def edge_gather(snd_hbm, rcv_hbm, send_hbm, rec_hbm, sa_out, sb_out,
                    d_out, sidx_all, ridx_all, buf_a, buf_b, dbuf,
                    sem_a, sem_b, sem_w):
        wid = lax.axis_index("s") * NC + lax.axis_index("c")
        e0 = wid * epw
        pltpu.sync_copy(send_hbm.at[pl.ds(e0, epw)], sidx_all)
        pltpu.sync_copy(rec_hbm.at[pl.ds(e0, epw)], ridx_all)

        def fire_gather(t, b):
            off = t * K
            pltpu.async_copy(snd_hbm.at[sidx_all.at[pl.ds(off, K)]],
                             buf_a.at[b], sem_a.at[b])
            pltpu.async_copy(rcv_hbm.at[ridx_all.at[pl.ds(off, K)]],
                             buf_b.at[b], sem_b.at[b])

        def wait_gather(t, b):
            off = t * K
            pltpu.make_async_copy(snd_hbm.at[sidx_all.at[pl.ds(off, K)]],
                                  buf_a.at[b], sem_a.at[b]).wait()
            pltpu.make_async_copy(rcv_hbm.at[ridx_all.at[pl.ds(off, K)]],
                                  buf_b.at[b], sem_b.at[b]).wait()

        def fire_write(t, b):
            base = e0 + t * K
            pltpu.async_copy(buf_a.at[b, :, pl.ds(0, H)],
                             sa_out.at[pl.ds(base, K)], sem_w.at[b])
            pltpu.async_copy(buf_b.at[b, :, pl.ds(0, H)],
                             sb_out.at[pl.ds(base, K)], sem_w.at[b])
            pltpu.async_copy(dbuf.at[b], d_out.at[pl.ds(base, K)], sem_w.at[b])

        def wait_write(t, b):
            base = e0 + t * K
            pltpu.make_async_copy(buf_a.at[b, :, pl.ds(0, H)],
                                  sa_out.at[pl.ds(base, K)], sem_w.at[b]).wait()
            pltpu.make_async_copy(buf_b.at[b, :, pl.ds(0, H)],
                                  sb_out.at[pl.ds(base, K)], sem_w.at[b]).wait()
            pltpu.make_async_copy(dbuf.at[b], d_out.at[pl.ds(base, K)],
                                  sem_w.at[b]).wait()

        fire_gather(0, 0)
        fire_gather(1, 1)

        def body(t, carry):
            b = t % 3
            fbuf = (t + 2) % 3

            @pl.when(t >= 1)
            def _():
                wait_write(t - 1, fbuf)

            @pl.when(t + 2 < ch)
            def _():
                fire_gather(t + 2, fbuf)

            wait_gather(t, b)

            def add_row(i, c):
                psl = pl.ds(H, 16)
                dbuf[b, i, pl.ds(0, 16)] = buf_a[b, i, psl] + buf_b[b, i, psl]
                return c
            lax.fori_loop(0, K, add_row, 0)
            fire_write(t, b)
            return carry

        lax.fori_loop(0, ch, body, 0)
        wait_write(ch - 1, (ch - 1) % 3)

    return edge_gather


@functools.cache
def _make_edge_scatter(ep):
    epw = ep // NW
    ng = epw // G
    tail = epw - ng * G

    @functools.partial(
        pl.kernel,
        mesh=_mesh,
        out_type=(
            jax.ShapeDtypeStruct((NC, N, H), jnp.float32),
            jax.ShapeDtypeStruct((NC, N, H), jnp.float32),
        ),
        scratch_types=(
            pltpu.VMEM((epw // K2, 1, K2), jnp.int32),
            pltpu.VMEM((2, G, H), jnp.float32),
            pltpu.VMEM_SHARED((N, H), jnp.float32),
            pltpu.SemaphoreType.DMA((2,)),
        ),
    )
    def edge_scatter(msg_hbm, pos_hbm, rec3_hbm, zeros_hbm,
                     out_m, out_p, ridx3, buf, acc, sem_r):
        c = lax.axis_index("c")
        s = lax.axis_index("s")
        wid = s * NC + c
        row0 = s * RPS
        is_last = s == NS - 1
        pltpu.sync_copy(rec3_hbm.at[pl.ds(wid * (epw // K2), epw // K2)],
                        ridx3)

        def scatter_phase(src_hbm, dst_hbm):
            pltpu.sync_copy(zeros_hbm.at[pl.ds(row0, RPS)],
                            acc.at[pl.ds(row0, RPS)])

            @pl.when(is_last)
            def _():
                pltpu.sync_copy(zeros_hbm.at[pl.ds(NS * RPS, NTAIL)],
                                acc.at[pl.ds(NS * RPS, NTAIL)])
            plsc.subcore_barrier()

            def fire_read(r, b):
                base = wid * epw + r * G
                pltpu.async_copy(src_hbm.at[pl.ds(base, G)], buf.at[b],
                                 sem_r.at[b])

            def wait_read(r, b):
                base = wid * epw + r * G
                pltpu.make_async_copy(src_hbm.at[pl.ds(base, G)], buf.at[b],
                                      sem_r.at[b]).wait()

            fire_read(0, 0)

            def chunk(r, carry):
                b = r % 2

                @pl.when(r + 1 < ng)
                def _():
                    fire_read(r + 1, 1 - b)

                wait_read(r, b)
                for j in range(G // K2):
                    pltpu.sync_copy(buf.at[b, pl.ds(j * K2, K2)],
                                    acc.at[ridx3.at[r * (G // K2) + j, 0]],
                                    add=True)
                return carry
            lax.fori_loop(0, ng, chunk, 0)
            if tail:
                base = wid * epw + ng * G
                pltpu.sync_copy(src_hbm.at[pl.ds(base, tail)],
                                buf.at[0, pl.ds(0, tail)])
                for j in range(tail // K2):
                    pltpu.sync_copy(buf.at[0, pl.ds(j * K2, K2)],
                                    acc.at[ridx3.at[ng * (G // K2) + j, 0]],
                                    add=True)
            plsc.subcore_barrier()
            pltpu.sync_copy(acc.at[pl.ds(row0, RPS)],
                            dst_hbm.at[c, pl.ds(row0, RPS)])

            @pl.when(is_last)
            def _():
                pltpu.sync_copy(acc.at[pl.ds(NS * RPS, NTAIL)],
                                dst_hbm.at[c, pl.ds(NS * RPS, NTAIL)])
            plsc.subcore_barrier()

        scatter_phase(msg_hbm, out_m)
        scatter_phase(pos_hbm, out_p)

    return edge_scatter


def _pack_pair(lo, hi):
    lo16 = jax.lax.bitcast_convert_type(lo.astype(jnp.bfloat16), jnp.uint16)
    hi16 = jax.lax.bitcast_convert_type(hi.astype(jnp.bfloat16), jnp.uint16)
    word = lo16.astype(jnp.uint32) | (hi16.astype(jnp.uint32) << 16)
    return jax.lax.bitcast_convert_type(word, jnp.float32)


def _node_pre_body(x_ref, pe_ref, ppad_ref, wx_ref, wp_ref, b_ref,
                   snd_ref, rcv_ref, ef_ref):
    x = x_ref[:]
    pe = pe_ref[:]
    sfeat = (x @ wx_ref[:, 0:2 * H] + pe @ wp_ref[:, 0:2 * H]
             + b_ref[:, 0:2 * H])
    snd_ref[:, 0:H] = _pack_pair(sfeat[:, 0:H], sfeat[:, H:2 * H])
    snd_ref[:, H:2 * H] = ppad_ref[:]
    rfeat = (x @ wx_ref[:, 2 * H:4 * H] + pe @ wp_ref[:, 2 * H:4 * H])
    rcv_ref[:, 0:H] = _pack_pair(rfeat[:, 0:H], rfeat[:, H:2 * H])
    rcv_ref[:, H:2 * H] = -ppad_ref[:]
    ef_ref[:] = (x @ wx_ref[:, 4 * H:6 * H] + pe @ wp_ref[:, 4 * H:6 * H]
                 + b_ref[:, 2 * H:4 * H])


def _unpack_pair(packed):
    bits = jax.lax.bitcast_convert_type(packed, jnp.uint32)
    lo = jax.lax.bitcast_convert_type(bits << 16, jnp.float32)
    hi = jax.lax.bitcast_convert_type(bits & jnp.uint32(0xFFFF0000),
                                      jnp.float32)
    return lo, hi


def _edge_mlp_body(sa_ref, sb_ref, d_ref, wrow_ref, brow_ref, w2_ref, p2_ref,
                   msg_ref, pmsg_ref):
    dvec = d_ref[:]
    dist = jnp.sqrt(jnp.sum(dvec * dvec, axis=1, keepdims=True))
    a1, a2 = _unpack_pair(sa_ref[:])
    b1_, b2_ = _unpack_pair(sb_ref[:])
    z1 = (a1 + b1_) + dist * wrow_ref[0:1, :]
    m1 = z1 * jax.nn.sigmoid(z1)
    mm = jnp.dot(m1, w2_ref[:], preferred_element_type=jnp.float32) \
        + brow_ref[0:1, :]
    msg_ref[:] = mm * jax.nn.sigmoid(mm)
    zp = (a2 + b2_) + dist * wrow_ref[1:2, :]
    p1 = jnp.tanh(zp)
    pp = jnp.dot(p1, p2_ref[:], preferred_element_type=jnp.float32) \
        + brow_ref[1:2, :]
    pmsg_ref[:] = jnp.tanh(pp)


def _update_body(ef_ref, pm1_ref, pm2_ref, pp1_ref, pp2_ref,
                 u1c_ref, u2_ref, ub2_ref, q1b_ref, q2_ref, qb2_ref,
                 upd_ref, updpe_ref):
    aggr = pm1_ref[0] + pm1_ref[1] + pm2_ref[0] + pm2_ref[1]
    u = ef_ref[:, 0:H] + jnp.dot(aggr, u1c_ref[:],
                                 preferred_element_type=jnp.float32)
    u = u * jax.nn.sigmoid(u)
    upd_ref[:] = jnp.dot(u, u2_ref[:],
                         preferred_element_type=jnp.float32) + ub2_ref[:]
    pos_aggr = pp1_ref[0] + pp1_ref[1] + pp2_ref[0] + pp2_ref[1]
    q = jnp.tanh(ef_ref[:, H:2 * H] + jnp.dot(pos_aggr, q1b_ref[:],
                                              preferred_element_type=jnp.float32))
    updpe_ref[:] = jnp.tanh(jnp.dot(q, q2_ref[:],
                                    preferred_element_type=jnp.float32)
                            + qb2_ref[:])


def kernel(x, pos, pe, edge_index, W1, b1, W2, b2, P1, pb1, P2, pb2,
           U1, ub1, U2, ub2, Q1, qb1, Q2, qb2):
    f32 = jnp.float32
    send = edge_index[0].astype(jnp.int32)
    rec = edge_index[1].astype(jnp.int32)
    ppad = jnp.concatenate([pos.astype(f32),
                            jnp.zeros((N, H - 3), f32)], axis=1)

    zH = jnp.zeros((H, H), f32)
    Wx = jnp.concatenate(
        [W1[0:H], zH, W1[2 * H:3 * H], zH, U1[0:H], zH], axis=1)
    Wp = jnp.concatenate(
        [W1[H:2 * H], P1[0:H], W1[3 * H:4 * H], P1[H:2 * H],
         U1[H:2 * H], Q1[0:H]], axis=1)
    bias = jnp.concatenate(
        [b1, pb1, ub1, qb1]).reshape(1, 4 * H)

    Tn = 2000
    snd_t, rcv_t, ef_t = pl.pallas_call(
        _node_pre_body,
        grid=(N // Tn,),
        in_specs=[
            pl.BlockSpec((Tn, H), lambda i: (i, 0)),
            pl.BlockSpec((Tn, H), lambda i: (i, 0)),
            pl.BlockSpec((Tn, H), lambda i: (i, 0)),
            pl.BlockSpec((H, 6 * H), lambda i: (0, 0)),
            pl.BlockSpec((H, 6 * H), lambda i: (0, 0)),
            pl.BlockSpec((1, 4 * H), lambda i: (0, 0)),
        ],
        out_specs=[
            pl.BlockSpec((Tn, 2 * H), lambda i: (i, 0)),
            pl.BlockSpec((Tn, 2 * H), lambda i: (i, 0)),
            pl.BlockSpec((Tn, 2 * H), lambda i: (i, 0)),
        ],
        out_shape=[
            jax.ShapeDtypeStruct((N, 2 * H), f32),
            jax.ShapeDtypeStruct((N, 2 * H), f32),
            jax.ShapeDtypeStruct((N, 2 * H), f32),
        ],
    )(x, pe, ppad, Wx, Wp, bias)

    wrow = jnp.stack([W1[4 * H], P1[2 * H]])
    brow = jnp.stack([b2, pb2])

    P = 2
    Eh = E // P
    gather_fn = _make_edge_gather(Eh)
    scatter_fn = _make_edge_scatter(Eh)
    Te = 2000
    zeros_nh = jnp.zeros((N, H), f32)

    partials = []
    for p in range(P):
        sl = slice(p * Eh, (p + 1) * Eh)
        sa_edge, sb_edge, d_edge = gather_fn(snd_t, rcv_t, send[sl], rec[sl])
        msg, pmsg = pl.pallas_call(
            _edge_mlp_body,
            grid=(Eh // Te,),
            in_specs=[
                pl.BlockSpec((Te, H), lambda i: (i, 0)),
                pl.BlockSpec((Te, H), lambda i: (i, 0)),
                pl.BlockSpec((Te, 16), lambda i: (i, 0)),
                pl.BlockSpec((2, H), lambda i: (0, 0)),
                pl.BlockSpec((2, H), lambda i: (0, 0)),
                pl.BlockSpec((H, H), lambda i: (0, 0)),
                pl.BlockSpec((H, H), lambda i: (0, 0)),
            ],
            out_specs=[
                pl.BlockSpec((Te, H), lambda i: (i, 0)),
                pl.BlockSpec((Te, H), lambda i: (i, 0)),
            ],
            out_shape=[
                jax.ShapeDtypeStruct((Eh, H), f32),
                jax.ShapeDtypeStruct((Eh, H), f32),
            ],
        )(sa_edge, sb_edge, d_edge, wrow, brow, W2, P2)
        rec3 = rec[sl].reshape(Eh // K2, 1, K2)
        pm, pp = scatter_fn(msg, pmsg, rec3, zeros_nh)
        partials.append((pm, pp))

    (pm1, pp1), (pm2, pp2) = partials
    upd, upd_pe = pl.pallas_call(
        _update_body,
        grid=(N // Tn,),
        in_specs=[
            pl.BlockSpec((Tn, 2 * H), lambda i: (i, 0)),
            pl.BlockSpec((NC, Tn, H), lambda i: (0, i, 0)),
            pl.BlockSpec((NC, Tn, H), lambda i: (0, i, 0)),
            pl.BlockSpec((NC, Tn, H), lambda i: (0, i, 0)),
            pl.BlockSpec((NC, Tn, H), lambda i: (0, i, 0)),
            pl.BlockSpec((H, H), lambda i: (0, 0)),
            pl.BlockSpec((H, H), lambda i: (0, 0)),
            pl.BlockSpec((1, H), lambda i: (0, 0)),
            pl.BlockSpec((H, H), lambda i: (0, 0)),
            pl.BlockSpec((H, H), lambda i: (0, 0)),
            pl.BlockSpec((1, H), lambda i: (0, 0)),
        ],
        out_specs=[
            pl.BlockSpec((Tn, H), lambda i: (i, 0)),
            pl.BlockSpec((Tn, H), lambda i: (i, 0)),
        ],
        out_shape=[
            jax.ShapeDtypeStruct((N, H), f32),
            jax.ShapeDtypeStruct((N, H), f32),
        ],
    )(ef_t, pm1, pm2, pp1, pp2, U1[2 * H:3 * H], U2, ub2.reshape(1, H),
      Q1[H:2 * H], Q2, qb2.reshape(1, H))

    return (upd, upd_pe)


# speedup vs baseline: 1.2561x; 1.0500x over previous
import functools

import jax
import jax.numpy as jnp
from jax import lax
from jax.experimental import pallas as pl
from jax.experimental.pallas import tpu as pltpu
from jax.experimental.pallas import tpu_sc as plsc

N = 10000
E = 320000
H = 128
W = 3 * H

NC = 2
NS = 16
NW = NC * NS
K = 40
K2 = 80
G = 80
RPS = 624
NTAIL = N - NS * RPS

_mesh = plsc.VectorSubcoreMesh(core_axis_name="c", subcore_axis_name="s")


@functools.cache
def _make_edge_gather(ep):
    epw = ep // NW
    ch = epw // K

    @functools.partial(
        pl.kernel,
        mesh=_mesh,
        out_type=(
            jax.ShapeDtypeStruct((ep, H), jnp.float32),
            jax.ShapeDtypeStruct((ep, H), jnp.float32),
            jax.ShapeDtypeStruct((ep, 16), jnp.float32),
        ),
        scratch_types=(
            pltpu.VMEM((epw,), jnp.int32),
            pltpu.VMEM((epw,), jnp.int32),
            pltpu.VMEM((3, K, 2 * H), jnp.float32),
            pltpu.VMEM((3, K, 2 * H), jnp.float32),
            pltpu.VMEM((3, K, 16), jnp.float32),
            pltpu.SemaphoreType.DMA((3,)),
            pltpu.SemaphoreType.DMA((3,)),
            pltpu.SemaphoreType.DMA((3,)),
        ),
    )
    def edge_gather(snd_hbm, rcv_hbm, send_hbm, rec_hbm, sa_out, sb_out,
                    d_out, sidx_all, ridx_all, buf_a, buf_b, dbuf,
                    sem_a, sem_b, sem_w):
        wid = lax.axis_index("s") * NC + lax.axis_index("c")
        e0 = wid * epw
        pltpu.sync_copy(send_hbm.at[pl.ds(e0, epw)], sidx_all)
        pltpu.sync_copy(rec_hbm.at[pl.ds(e0, epw)], ridx_all)

        def fire_gather(t, b):
            off = t * K
            pltpu.async_copy(snd_hbm.at[sidx_all.at[pl.ds(off, K)]],
                             buf_a.at[b], sem_a.at[b])
            pltpu.async_copy(rcv_hbm.at[ridx_all.at[pl.ds(off, K)]],
                             buf_b.at[b], sem_b.at[b])

        def wait_gather(t, b):
            off = t * K
            pltpu.make_async_copy(snd_hbm.at[sidx_all.at[pl.ds(off, K)]],
                                  buf_a.at[b], sem_a.at[b]).wait()
            pltpu.make_async_copy(rcv_hbm.at[ridx_all.at[pl.ds(off, K)]],
                                  buf_b.at[b], sem_b.at[b]).wait()

        def fire_write(t, b):
            base = e0 + t * K
            pltpu.async_copy(buf_a.at[b, :, pl.ds(0, H)],
                             sa_out.at[pl.ds(base, K)], sem_w.at[b])
            pltpu.async_copy(buf_b.at[b, :, pl.ds(0, H)],
                             sb_out.at[pl.ds(base, K)], sem_w.at[b])
            pltpu.async_copy(dbuf.at[b], d_out.at[pl.ds(base, K)], sem_w.at[b])

        def wait_write(t, b):
            base = e0 + t * K
            pltpu.make_async_copy(buf_a.at[b, :, pl.ds(0, H)],
                                  sa_out.at[pl.ds(base, K)], sem_w.at[b]).wait()
            pltpu.make_async_copy(buf_b.at[b, :, pl.ds(0, H)],
                                  sb_out.at[pl.ds(base, K)], sem_w.at[b]).wait()
            pltpu.make_async_copy(dbuf.at[b], d_out.at[pl.ds(base, K)],
                                  sem_w.at[b]).wait()

        fire_gather(0, 0)
        fire_gather(1, 1)

        def body(t, carry):
            b = t % 3
            fbuf = (t + 2) % 3

            @pl.when(t >= 1)
            def _():
                wait_write(t - 1, fbuf)

            @pl.when(t + 2 < ch)
            def _():
                fire_gather(t + 2, fbuf)

            wait_gather(t, b)

            def add_row(i, c):
                psl = pl.ds(H, 16)
                dbuf[b, i, pl.ds(0, 16)] = buf_a[b, i, psl] + buf_b[b, i, psl]
                return c
            lax.fori_loop(0, K, add_row, 0)
            fire_write(t, b)
            return carry

        lax.fori_loop(0, ch, body, 0)
        wait_write(ch - 1, (ch - 1) % 3)

    return edge_gather


@functools.cache
def _make_edge_scatter(ep):
    epw = ep // NS
    ng = epw // G

    @functools.partial(
        pl.kernel,
        mesh=_mesh,
        out_type=(
            jax.ShapeDtypeStruct((N, H), jnp.float32),
            jax.ShapeDtypeStruct((N, H), jnp.float32),
        ),
        scratch_types=(
            pltpu.VMEM((epw // K2, 1, K2), jnp.int32),
            pltpu.VMEM((2, G, H), jnp.float32),
            pltpu.VMEM_SHARED((N, H), jnp.float32),
            pltpu.SemaphoreType.DMA((2,)),
        ),
    )
    def edge_scatter(msg_hbm, pos_hbm, rec3_hbm, zeros_hbm,
                     out_m, out_p, ridx3, buf, acc, sem_r):
        c = lax.axis_index("c")
        s = lax.axis_index("s")
        row0 = s * RPS
        is_last = s == NS - 1
        pltpu.sync_copy(rec3_hbm.at[pl.ds(s * (epw // K2), epw // K2)],
                        ridx3)

        pltpu.sync_copy(zeros_hbm.at[pl.ds(row0, RPS)],
                        acc.at[pl.ds(row0, RPS)])

        @pl.when(is_last)
        def _():
            pltpu.sync_copy(zeros_hbm.at[pl.ds(NS * RPS, NTAIL)],
                            acc.at[pl.ds(NS * RPS, NTAIL)])
        plsc.subcore_barrier()

        def scatter_loop(src_hbm):
            def fire_read(r, b):
                base = s * epw + r * G
                pltpu.async_copy(src_hbm.at[pl.ds(base, G)], buf.at[b],
                                 sem_r.at[b])

            def wait_read(r, b):
                base = s * epw + r * G
                pltpu.make_async_copy(src_hbm.at[pl.ds(base, G)], buf.at[b],
                                      sem_r.at[b]).wait()

            fire_read(0, 0)

            def chunk(r, carry):
                b = r % 2

                @pl.when(r + 1 < ng)
                def _():
                    fire_read(r + 1, 1 - b)

                wait_read(r, b)
                for j in range(G // K2):
                    pltpu.sync_copy(buf.at[b, pl.ds(j * K2, K2)],
                                    acc.at[ridx3.at[r * (G // K2) + j, 0]],
                                    add=True)
                return carry
            lax.fori_loop(0, ng, chunk, 0)

        @pl.when(c == 0)
        def _():
            scatter_loop(msg_hbm)

        @pl.when(c == 1)
        def _():
            scatter_loop(pos_hbm)

        plsc.subcore_barrier()

        def copy_out(dst_hbm):
            pltpu.sync_copy(acc.at[pl.ds(row0, RPS)],
                            dst_hbm.at[pl.ds(row0, RPS)])

            @pl.when(is_last)
            def _():
                pltpu.sync_copy(acc.at[pl.ds(NS * RPS, NTAIL)],
                                dst_hbm.at[pl.ds(NS * RPS, NTAIL)])

        @pl.when(c == 0)
        def _():
            copy_out(out_m)

        @pl.when(c == 1)
        def _():
            copy_out(out_p)
        plsc.subcore_barrier()

    return edge_scatter


def _pack_pair(lo, hi):
    lo16 = jax.lax.bitcast_convert_type(lo.astype(jnp.bfloat16), jnp.uint16)
    hi16 = jax.lax.bitcast_convert_type(hi.astype(jnp.bfloat16), jnp.uint16)
    word = lo16.astype(jnp.uint32) | (hi16.astype(jnp.uint32) << 16)
    return jax.lax.bitcast_convert_type(word, jnp.float32)


def _node_pre_body(x_ref, pe_ref, ppad_ref, wx_ref, wp_ref, b_ref,
                   snd_ref, rcv_ref, ef_ref):
    x = x_ref[:]
    pe = pe_ref[:]
    sfeat = (x @ wx_ref[:, 0:2 * H] + pe @ wp_ref[:, 0:2 * H]
             + b_ref[:, 0:2 * H])
    snd_ref[:, 0:H] = _pack_pair(sfeat[:, 0:H], sfeat[:, H:2 * H])
    snd_ref[:, H:2 * H] = ppad_ref[:]
    rfeat = (x @ wx_ref[:, 2 * H:4 * H] + pe @ wp_ref[:, 2 * H:4 * H])
    rcv_ref[:, 0:H] = _pack_pair(rfeat[:, 0:H], rfeat[:, H:2 * H])
    rcv_ref[:, H:2 * H] = -ppad_ref[:]
    ef_ref[:] = (x @ wx_ref[:, 4 * H:6 * H] + pe @ wp_ref[:, 4 * H:6 * H]
                 + b_ref[:, 2 * H:4 * H])


def _unpack_pair(packed):
    bits = jax.lax.bitcast_convert_type(packed, jnp.uint32)
    lo = jax.lax.bitcast_convert_type(bits << 16, jnp.float32)
    hi = jax.lax.bitcast_convert_type(bits & jnp.uint32(0xFFFF0000),
                                      jnp.float32)
    return lo, hi


def _edge_mlp_body(sa_ref, sb_ref, d_ref, wrow_ref, brow_ref, w2_ref, p2_ref,
                   msg_ref, pmsg_ref):
    dvec = d_ref[:]
    dist = jnp.sqrt(jnp.sum(dvec * dvec, axis=1, keepdims=True))
    a1, a2 = _unpack_pair(sa_ref[:])
    b1_, b2_ = _unpack_pair(sb_ref[:])
    z1 = (a1 + b1_) + dist * wrow_ref[0:1, :]
    m1 = z1 * jax.nn.sigmoid(z1)
    mm = jnp.dot(m1, w2_ref[:], preferred_element_type=jnp.float32) \
        + brow_ref[0:1, :]
    msg_ref[:] = mm * jax.nn.sigmoid(mm)
    zp = (a2 + b2_) + dist * wrow_ref[1:2, :]
    p1 = jnp.tanh(zp)
    pp = jnp.dot(p1, p2_ref[:], preferred_element_type=jnp.float32) \
        + brow_ref[1:2, :]
    pmsg_ref[:] = jnp.tanh(pp)


def _update_body(ef_ref, pm1_ref, pm2_ref, pp1_ref, pp2_ref,
                 u1c_ref, u2_ref, ub2_ref, q1b_ref, q2_ref, qb2_ref,
                 upd_ref, updpe_ref):
    aggr = pm1_ref[:] + pm2_ref[:]
    u = ef_ref[:, 0:H] + jnp.dot(aggr, u1c_ref[:],
                                 preferred_element_type=jnp.float32)
    u = u * jax.nn.sigmoid(u)
    upd_ref[:] = jnp.dot(u, u2_ref[:],
                         preferred_element_type=jnp.float32) + ub2_ref[:]
    pos_aggr = pp1_ref[:] + pp2_ref[:]
    q = jnp.tanh(ef_ref[:, H:2 * H] + jnp.dot(pos_aggr, q1b_ref[:],
                                              preferred_element_type=jnp.float32))
    updpe_ref[:] = jnp.tanh(jnp.dot(q, q2_ref[:],
                                    preferred_element_type=jnp.float32)
                            + qb2_ref[:])


def kernel(x, pos, pe, edge_index, W1, b1, W2, b2, P1, pb1, P2, pb2,
           U1, ub1, U2, ub2, Q1, qb1, Q2, qb2):
    f32 = jnp.float32
    send = edge_index[0].astype(jnp.int32)
    rec = edge_index[1].astype(jnp.int32)
    ppad = jnp.concatenate([pos.astype(f32),
                            jnp.zeros((N, H - 3), f32)], axis=1)

    zH = jnp.zeros((H, H), f32)
    Wx = jnp.concatenate(
        [W1[0:H], zH, W1[2 * H:3 * H], zH, U1[0:H], zH], axis=1)
    Wp = jnp.concatenate(
        [W1[H:2 * H], P1[0:H], W1[3 * H:4 * H], P1[H:2 * H],
         U1[H:2 * H], Q1[0:H]], axis=1)
    bias = jnp.concatenate(
        [b1, pb1, ub1, qb1]).reshape(1, 4 * H)

    Tn = 2000
    snd_t, rcv_t, ef_t = pl.pallas_call(
        _node_pre_body,
        grid=(N // Tn,),
        in_specs=[
            pl.BlockSpec((Tn, H), lambda i: (i, 0)),
            pl.BlockSpec((Tn, H), lambda i: (i, 0)),
            pl.BlockSpec((Tn, H), lambda i: (i, 0)),
            pl.BlockSpec((H, 6 * H), lambda i: (0, 0)),
            pl.BlockSpec((H, 6 * H), lambda i: (0, 0)),
            pl.BlockSpec((1, 4 * H), lambda i: (0, 0)),
        ],
        out_specs=[
            pl.BlockSpec((Tn, 2 * H), lambda i: (i, 0)),
            pl.BlockSpec((Tn, 2 * H), lambda i: (i, 0)),
            pl.BlockSpec((Tn, 2 * H), lambda i: (i, 0)),
        ],
        out_shape=[
            jax.ShapeDtypeStruct((N, 2 * H), f32),
            jax.ShapeDtypeStruct((N, 2 * H), f32),
            jax.ShapeDtypeStruct((N, 2 * H), f32),
        ],
    )(x, pe, ppad, Wx, Wp, bias)

    wrow = jnp.stack([W1[4 * H], P1[2 * H]])
    brow = jnp.stack([b2, pb2])

    P = 2
    Eh = E // P
    gather_fn = _make_edge_gather(Eh)
    scatter_fn = _make_edge_scatter(Eh)
    Te = 2000
    zeros_nh = jnp.zeros((N, H), f32)

    partials = []
    for p in range(P):
        sl = slice(p * Eh, (p + 1) * Eh)
        sa_edge, sb_edge, d_edge = gather_fn(snd_t, rcv_t, send[sl], rec[sl])
        msg, pmsg = pl.pallas_call(
            _edge_mlp_body,
            grid=(Eh // Te,),
            in_specs=[
                pl.BlockSpec((Te, H), lambda i: (i, 0)),
                pl.BlockSpec((Te, H), lambda i: (i, 0)),
                pl.BlockSpec((Te, 16), lambda i: (i, 0)),
                pl.BlockSpec((2, H), lambda i: (0, 0)),
                pl.BlockSpec((2, H), lambda i: (0, 0)),
                pl.BlockSpec((H, H), lambda i: (0, 0)),
                pl.BlockSpec((H, H), lambda i: (0, 0)),
            ],
            out_specs=[
                pl.BlockSpec((Te, H), lambda i: (i, 0)),
                pl.BlockSpec((Te, H), lambda i: (i, 0)),
            ],
            out_shape=[
                jax.ShapeDtypeStruct((Eh, H), f32),
                jax.ShapeDtypeStruct((Eh, H), f32),
            ],
        )(sa_edge, sb_edge, d_edge, wrow, brow, W2, P2)
        rec3 = rec[sl].reshape(Eh // K2, 1, K2)
        pm, pp = scatter_fn(msg, pmsg, rec3, zeros_nh)
        partials.append((pm, pp))

    (pm1, pp1), (pm2, pp2) = partials
    upd, upd_pe = pl.pallas_call(
        _update_body,
        grid=(N // Tn,),
        in_specs=[
            pl.BlockSpec((Tn, 2 * H), lambda i: (i, 0)),
            pl.BlockSpec((Tn, H), lambda i: (i, 0)),
            pl.BlockSpec((Tn, H), lambda i: (i, 0)),
            pl.BlockSpec((Tn, H), lambda i: (i, 0)),
            pl.BlockSpec((Tn, H), lambda i: (i, 0)),
            pl.BlockSpec((H, H), lambda i: (0, 0)),
            pl.BlockSpec((H, H), lambda i: (0, 0)),
            pl.BlockSpec((1, H), lambda i: (0, 0)),
            pl.BlockSpec((H, H), lambda i: (0, 0)),
            pl.BlockSpec((H, H), lambda i: (0, 0)),
            pl.BlockSpec((1, H), lambda i: (0, 0)),
        ],
        out_specs=[
            pl.BlockSpec((Tn, H), lambda i: (i, 0)),
            pl.BlockSpec((Tn, H), lambda i: (i, 0)),
        ],
        out_shape=[
            jax.ShapeDtypeStruct((N, H), f32),
            jax.ShapeDtypeStruct((N, H), f32),
        ],
    )(ef_t, pm1, pm2, pp1, pp2, U1[2 * H:3 * H], U2, ub2.reshape(1, H),
      Q1[H:2 * H], Q2, qb2.reshape(1, H))

    return (upd, upd_pe)
